# baseline (speedup 1.0000x reference)
import jax
import jax.numpy as jnp
from jax.experimental import pallas as pl
from jax.experimental.pallas import tpu as pltpu

_B, _S, _D, _H, _E, _K = 32, 4, 768, 256, 8, 2
_EPS_NORM = 1e-6


def _moe_kernel(x_ref, nw_ref, gwt_ref, w1_ref, w2_ref, w3_ref, out_ref,
                xn_ref, c_ref):
    e = pl.program_id(0)

    @pl.when(e == 0)
    def _prologue():
        x = x_ref[...]
        xn = x * jax.lax.rsqrt(
            jnp.mean(x * x, axis=-1, keepdims=True) + _EPS_NORM)
        xn = xn * nw_ref[...]
        xn_ref[...] = xn
        logits = jnp.dot(xn, gwt_ref[...], preferred_element_type=jnp.float32)
        p = jax.nn.softmax(logits, axis=-1)
        iota = jax.lax.broadcasted_iota(jnp.int32, p.shape, 1)
        m1 = jnp.max(p, axis=-1, keepdims=True)
        i1 = jnp.min(jnp.where(p >= m1, iota, _E), axis=-1, keepdims=True)
        one1 = iota == i1
        p2 = jnp.where(one1, -1.0, p)
        m2 = jnp.max(p2, axis=-1, keepdims=True)
        i2 = jnp.min(jnp.where(p2 >= m2, iota, _E), axis=-1, keepdims=True)
        one2 = iota == i2
        c_ref[...] = jnp.where(one1 | one2, p, 0.0) / (m1 + m2 + 1e-10)

    xn = xn_ref[...]
    h1 = jnp.dot(xn, w1_ref[0], preferred_element_type=jnp.float32)
    h2 = jnp.dot(xn, w2_ref[0], preferred_element_type=jnp.float32)
    hid = (h1 * jax.lax.logistic(h1)) * h2
    oe = jnp.dot(hid, w3_ref[0], preferred_element_type=jnp.float32)
    c = c_ref[...]
    lane = jax.lax.broadcasted_iota(jnp.int32, c.shape, 1)
    ce = jnp.sum(jnp.where(lane == e, c, 0.0), axis=-1, keepdims=True)
    contrib = ce * oe

    @pl.when(e == 0)
    def _init():
        out_ref[...] = contrib

    @pl.when(e > 0)
    def _accum():
        out_ref[...] += contrib


def kernel(x, norm_weight, gate_w, w1, w2, w3):
    b, s, d = x.shape
    t = b * s
    x_flat = x.reshape(t, d)
    nw = norm_weight.reshape(1, d)
    gwt = gate_w.T
    out = pl.pallas_call(
        _moe_kernel,
        grid=(_E,),
        in_specs=[
            pl.BlockSpec((t, d), lambda e: (0, 0)),
            pl.BlockSpec((1, d), lambda e: (0, 0)),
            pl.BlockSpec((d, _E), lambda e: (0, 0)),
            pl.BlockSpec((1, _D, _H), lambda e: (e, 0, 0)),
            pl.BlockSpec((1, _D, _H), lambda e: (e, 0, 0)),
            pl.BlockSpec((1, _H, _D), lambda e: (e, 0, 0)),
        ],
        out_specs=pl.BlockSpec((t, d), lambda e: (0, 0)),
        out_shape=jax.ShapeDtypeStruct((t, d), jnp.float32),
        scratch_shapes=[
            pltpu.VMEM((t, d), jnp.float32),
            pltpu.VMEM((t, _E), jnp.float32),
        ],
    )(x_flat, nw, gwt, w1, w2, w3)
    return out.reshape(b, s, d)
